# Initial kernel scaffold; baseline (speedup 1.0000x reference)
import jax
import jax.numpy as jnp
from jax.experimental import pallas as pl


def kernel(neighbors, table):
    raise NotImplementedError("write your pallas kernel here")


# speedup vs baseline: 1.5926x; 1.5926x over previous
import functools

import jax
import jax.numpy as jnp
from jax import lax
from jax.experimental import pallas as pl
from jax.experimental.pallas import tpu as pltpu
from jax.experimental.pallas import tpu_sc as plsc

N_NODES = 10000
NUM_SAMPLE = 32
FEAT_DIM = 128

NUM_WORKERS = 32
NODES_PER_WORKER = 320
N_PAD = NUM_WORKERS * NODES_PER_WORKER
CHUNK_NODES = 2
CHUNK_IDX = CHUNK_NODES * NUM_SAMPLE
NUM_CHUNKS = NODES_PER_WORKER // CHUNK_NODES
LANES = 16
D_BLOCKS = FEAT_DIM // LANES
SCALE = 1.0 / NUM_SAMPLE


def _agg_body(idx_hbm, table_hbm, out_hbm, idx_v, buf0, buf1, out_v,
              sem0, sem1):
    wid = lax.axis_index("s") * 2 + lax.axis_index("c")

    pltpu.sync_copy(idx_hbm.at[wid], idx_v)

    bufs = (buf0, buf1)
    sems = (sem0, sem1)

    def start(c, b):
        pltpu.async_copy(table_hbm.at[idx_v.at[c]], bufs[b], sems[b])

    def wait(b):
        pltpu.make_async_copy(table_hbm.at[pl.ds(0, CHUNK_IDX)], bufs[b],
                              sems[b]).wait()

    def reduce_chunk(c, b):
        buf = bufs[b]
        for k in range(CHUNK_NODES):
            base = k * NUM_SAMPLE
            for d in range(D_BLOCKS):
                sl = pl.ds(d * LANES, LANES)
                acc = buf[base, sl]
                for j in range(1, NUM_SAMPLE):
                    acc = acc + buf[base + j, sl]
                out_v[c * CHUNK_NODES + k, sl] = acc * SCALE

    start(0, 0)
    start(1, 1)

    def body(i, carry):
        c0 = i * 2
        wait(0)
        reduce_chunk(c0, 0)

        @pl.when(c0 + 2 < NUM_CHUNKS)
        def _():
            start(c0 + 2, 0)

        wait(1)
        reduce_chunk(c0 + 1, 1)

        @pl.when(c0 + 3 < NUM_CHUNKS)
        def _():
            start(c0 + 3, 1)

        return carry

    lax.fori_loop(0, NUM_CHUNKS // 2, body, 0)

    pltpu.sync_copy(out_v, out_hbm.at[pl.ds(wid * NODES_PER_WORKER,
                                            NODES_PER_WORKER)])


_mesh = plsc.VectorSubcoreMesh(core_axis_name="c", subcore_axis_name="s")

_agg = functools.partial(
    pl.kernel,
    out_type=jax.ShapeDtypeStruct((N_PAD, FEAT_DIM), jnp.float32),
    mesh=_mesh,
    scratch_types=[
        pltpu.VMEM((NUM_CHUNKS, CHUNK_IDX), jnp.int32),
        pltpu.VMEM((CHUNK_IDX, FEAT_DIM), jnp.float32),
        pltpu.VMEM((CHUNK_IDX, FEAT_DIM), jnp.float32),
        pltpu.VMEM((NODES_PER_WORKER, FEAT_DIM), jnp.float32),
        pltpu.SemaphoreType.DMA,
        pltpu.SemaphoreType.DMA,
    ],
)(_agg_body)


@jax.jit
def kernel(neighbors, table):
    nbr = neighbors.astype(jnp.int32)
    nbr = jnp.pad(nbr, ((0, N_PAD - N_NODES), (0, 0)))
    idx = nbr.reshape(NUM_WORKERS, NUM_CHUNKS, CHUNK_IDX)
    out = _agg(idx, table)
    return out[:N_NODES]
